# baseline (speedup 1.0000x reference)
import functools

import jax
import jax.numpy as jnp
from jax import lax
from jax.experimental import pallas as pl
from jax.experimental.pallas import tpu as pltpu
from jax.experimental.pallas import tpu_sc as plsc

EMBED = 64
LANES = 16
NENT = 1000000
TAILN = 64
NMAIN = NENT - TAILN
HALF_OFF = NMAIN // 2
TBLK = 768
NBLK = HALF_OFF // TBLK
HALF = 256


CCHUNK = 128
NCHUNKS = HALF_OFF // CCHUNK


@functools.lru_cache(maxsize=None)
def _tpose(nc, ns):
    nw = nc * ns
    iters = -(-NCHUNKS // nw)
    mesh = plsc.VectorSubcoreMesh(core_axis_name="c", subcore_axis_name="s")

    @functools.partial(
        pl.kernel,
        out_type=jax.ShapeDtypeStruct((HALF_OFF, 128), jnp.float32),
        mesh=mesh,
        compiler_params=pltpu.CompilerParams(
            needs_layout_passes=False, use_tc_tiling_on_sc=True
        ),
        scratch_types=[
            pltpu.VMEM((EMBED, CCHUNK), jnp.float32),
            pltpu.VMEM((EMBED, CCHUNK), jnp.float32),
            pltpu.VMEM((CCHUNK, 128), jnp.float32),
        ],
    )
    def tposer(entT, out, sa, sb, ob):
        c = lax.axis_index("c")
        s = lax.axis_index("s")
        wid = s * nc + c

        def body(t, carry):
            idx = wid + nw * t

            @pl.when(idx < NCHUNKS)
            def _chunk():
                p0 = pl.multiple_of(idx * CCHUNK, 128)
                pltpu.sync_copy(entT.at[:, pl.ds(p0, CCHUNK)], sa)
                pltpu.sync_copy(entT.at[:, pl.ds(p0 + HALF_OFF, CCHUNK)], sb)

                def tbody(p, carry2):
                    pv = jnp.full((LANES,), p, jnp.int32)
                    for cg in range(EMBED // LANES):
                        didx = cg * LANES + lax.iota(jnp.int32, LANES)
                        ob[p, pl.ds(cg * LANES, LANES)] = plsc.load_gather(
                            sa, [didx, pv]
                        )
                        ob[p, pl.ds(EMBED + cg * LANES, LANES)] = plsc.load_gather(
                            sb, [didx, pv]
                        )
                    return carry2

                lax.fori_loop(0, CCHUNK, tbody, 0)
                pltpu.sync_copy(ob, out.at[pl.ds(p0, CCHUNK), :])

            return carry

        lax.fori_loop(0, iters, body, 0)

    return tposer


def _sqrt16(a):
    i = lax.bitcast_convert_type(a, jnp.int32)
    y = lax.bitcast_convert_type(jnp.int32(0x5F3759DF) - (i >> 1), jnp.float32)
    for _ in range(3):
        y = y * (1.5 - (0.5 * a * y) * y)
    return a * y


@functools.lru_cache(maxsize=None)
def _build(nc, ns, bpt):
    ngroups = HALF // LANES
    mesh = plsc.VectorSubcoreMesh(core_axis_name="c", subcore_axis_name="s")

    @functools.partial(
        pl.kernel,
        out_type=jax.ShapeDtypeStruct((nc * ns * bpt,), jnp.float32),
        mesh=mesh,
        compiler_params=pltpu.CompilerParams(
            needs_layout_passes=False, use_tc_tiling_on_sc=False
        ),
        scratch_types=[
            pltpu.VMEM((bpt,), jnp.int32),
            pltpu.VMEM((bpt,), jnp.int32),
            pltpu.VMEM((bpt,), jnp.int32),
            pltpu.VMEM((HALF, 128), jnp.float32),
            pltpu.VMEM((HALF, EMBED), jnp.float32),
            pltpu.VMEM((HALF, 128), jnp.float32),
            pltpu.VMEM((TAILN, EMBED), jnp.float32),
            pltpu.VMEM((bpt,), jnp.float32),
            pltpu.VMEM((bpt,), jnp.int32),
            pltpu.VMEM((bpt,), jnp.int32),
            pltpu.SemaphoreType.DMA,
        ],
    )
    def trans_e(ent2, rel, corner, headi, reli, taili, out,
                hidx, ridx, tidx, hrows, rrows, trows, cbuf, outv, hpi, tpi, sem):
        c = lax.axis_index("c")
        s = lax.axis_index("s")
        wid = s * nc + c
        base = wid * bpt

        pltpu.sync_copy(headi.at[pl.ds(base, bpt)], hidx)
        pltpu.sync_copy(reli.at[pl.ds(base, bpt)], ridx)
        pltpu.sync_copy(taili.at[pl.ds(base, bpt)], tidx)
        pltpu.sync_copy(corner, cbuf)

        def ibody(g, carry):
            sl = pl.ds(g * LANES, LANES)
            for src, dst in ((hidx, hpi), (tidx, tpi)):
                e = src[sl]
                pi = jnp.where(e >= HALF_OFF, e - HALF_OFF, e)
                dst[sl] = jnp.where(e >= NMAIN, 0, pi)
            return carry

        lax.fori_loop(0, bpt // LANES, ibody, 0)

        for half in range(bpt // HALF):
            hbase = half * HALF
            copies = []
            for j in range(HALF // 128):
                isl = pl.ds(hbase + j * 128, 128)
                dsl = pl.ds(j * 128, 128)
                copies.append(pltpu.async_copy(ent2.at[hpi.at[isl]], hrows.at[dsl], sem))
                copies.append(pltpu.async_copy(rel.at[ridx.at[isl]], rrows.at[dsl], sem))
                copies.append(pltpu.async_copy(ent2.at[tpi.at[isl]], trows.at[dsl], sem))
            for cp in copies:
                cp.wait()

            def pbody(g, carry):
                sl = pl.ds(hbase + g * LANES, LANES)
                rows16 = g * LANES + lax.iota(jnp.int32, LANES)
                for idxref, rowsref in ((hidx, hrows), (tidx, trows)):
                    e = idxref[sl]
                    m = e >= NMAIN
                    n = plsc.all_reduce_population_count(m)

                    @pl.when(jnp.max(n) > 0)
                    def _patch():
                        ce = (e - NMAIN) & (TAILN - 1)
                        off = jnp.where(e >= HALF_OFF, EMBED, 0)
                        for jj in range(EMBED):
                            cj = jnp.full((LANES,), jj, jnp.int32)
                            cv = plsc.load_gather(cbuf, [ce, cj])
                            plsc.store_scatter(rowsref, [rows16, off + cj], cv, mask=m)

                return carry

            lax.fori_loop(0, ngroups, pbody, 0)

            def gbody(g, carry):
                rows16 = g * LANES + lax.iota(jnp.int32, LANES)
                sl = pl.ds(hbase + g * LANES, LANES)
                hoff = jnp.where(hidx[sl] >= HALF_OFF, EMBED, 0)
                toff = jnp.where(tidx[sl] >= HALF_OFF, EMBED, 0)

                def jbody(j, acc):
                    for u in range(4):
                        col = jnp.full((LANES,), j * 4 + u, jnp.int32)
                        hv = plsc.load_gather(hrows, [rows16, hoff + col])
                        rv = plsc.load_gather(rrows, [rows16, col])
                        tv = plsc.load_gather(trows, [rows16, toff + col])
                        d = (hv + rv) - tv
                        acc = acc + d * d
                    return acc

                acc = lax.fori_loop(
                    0, EMBED // 4, jbody, jnp.zeros((LANES,), jnp.float32)
                )
                outv[pl.ds(hbase + g * LANES, LANES)] = _sqrt16(acc)
                return carry

            lax.fori_loop(0, ngroups, gbody, 0)

        pltpu.sync_copy(outv, out.at[pl.ds(base, bpt)])

    return trans_e


def kernel(entity_embeddings, relation_embeddings, head, relation, tail):
    info = plsc.get_sparse_core_info()
    nc, ns = info.num_cores, info.num_subcores
    batch = head.shape[0]
    bpt = batch // (nc * ns)
    entT = entity_embeddings.T
    ent2 = _tpose(nc, ns)(entT)
    corner = entity_embeddings[NMAIN:, :]
    out = _build(nc, ns, bpt)(
        ent2, relation_embeddings, corner, head, relation, tail
    )
    return out


# speedup vs baseline: 5.1654x; 5.1654x over previous
import functools

import jax
import jax.numpy as jnp
from jax import lax
from jax.experimental import pallas as pl
from jax.experimental.pallas import tpu as pltpu
from jax.experimental.pallas import tpu_sc as plsc

EMBED = 64
LANES = 16
NENT = 1000000
TAILN = 64
NMAIN = NENT - TAILN
HALF_OFF = NMAIN // 2
TBLK = 2688
NBLK = HALF_OFF // TBLK
HALF = 256


def _tpose_kernel(a_ref, b_ref, eye_ref, o_ref):
    dn = (((0,), (0,)), ((), ()))
    at = lax.dot_general(a_ref[...], eye_ref[...], dn,
                         preferred_element_type=jnp.float32)
    bt = lax.dot_general(b_ref[...], eye_ref[...], dn,
                         preferred_element_type=jnp.float32)
    o_ref[...] = jnp.concatenate([at, bt], axis=1)


@functools.lru_cache(maxsize=None)
def _tpose(nc, ns):
    del nc, ns
    return pl.pallas_call(
        _tpose_kernel,
        grid=(NBLK,),
        in_specs=[
            pl.BlockSpec((EMBED, TBLK), lambda j: (0, j)),
            pl.BlockSpec((EMBED, TBLK), lambda j: (0, j + NBLK)),
            pl.BlockSpec((EMBED, EMBED), lambda j: (0, 0)),
        ],
        out_specs=pl.BlockSpec((TBLK, 128), lambda j: (j, 0)),
        out_shape=jax.ShapeDtypeStruct((HALF_OFF, 128), jnp.float32),
    )


def _sqrt16(a):
    i = lax.bitcast_convert_type(a, jnp.int32)
    y = lax.bitcast_convert_type(jnp.int32(0x5F3759DF) - (i >> 1), jnp.float32)
    for _ in range(3):
        y = y * (1.5 - (0.5 * a * y) * y)
    return a * y


@functools.lru_cache(maxsize=None)
def _build(nc, ns, bpt):
    ngroups = HALF // LANES
    mesh = plsc.VectorSubcoreMesh(core_axis_name="c", subcore_axis_name="s")

    @functools.partial(
        pl.kernel,
        out_type=jax.ShapeDtypeStruct((nc * ns * bpt,), jnp.float32),
        mesh=mesh,
        compiler_params=pltpu.CompilerParams(
            needs_layout_passes=False, use_tc_tiling_on_sc=False
        ),
        scratch_types=[
            pltpu.VMEM((bpt,), jnp.int32),
            pltpu.VMEM((bpt,), jnp.int32),
            pltpu.VMEM((bpt,), jnp.int32),
            pltpu.VMEM((HALF, 128), jnp.float32),
            pltpu.VMEM((HALF, EMBED), jnp.float32),
            pltpu.VMEM((HALF, 128), jnp.float32),
            pltpu.VMEM((TAILN, EMBED), jnp.float32),
            pltpu.VMEM((bpt,), jnp.float32),
            pltpu.VMEM((bpt,), jnp.int32),
            pltpu.VMEM((bpt,), jnp.int32),
            pltpu.SemaphoreType.DMA,
        ],
    )
    def trans_e(ent2, rel, corner, headi, reli, taili, out,
                hidx, ridx, tidx, hrows, rrows, trows, cbuf, outv, hpi, tpi, sem):
        c = lax.axis_index("c")
        s = lax.axis_index("s")
        wid = s * nc + c
        base = wid * bpt

        pltpu.sync_copy(headi.at[pl.ds(base, bpt)], hidx)
        pltpu.sync_copy(reli.at[pl.ds(base, bpt)], ridx)
        pltpu.sync_copy(taili.at[pl.ds(base, bpt)], tidx)
        pltpu.sync_copy(corner, cbuf)

        def ibody(g, carry):
            sl = pl.ds(g * LANES, LANES)
            for src, dst in ((hidx, hpi), (tidx, tpi)):
                e = src[sl]
                pi = jnp.where(e >= HALF_OFF, e - HALF_OFF, e)
                dst[sl] = jnp.where(e >= NMAIN, 0, pi)
            return carry

        lax.fori_loop(0, bpt // LANES, ibody, 0)

        for half in range(bpt // HALF):
            hbase = half * HALF
            copies = []
            for j in range(HALF // 128):
                isl = pl.ds(hbase + j * 128, 128)
                dsl = pl.ds(j * 128, 128)
                copies.append(pltpu.async_copy(ent2.at[hpi.at[isl]], hrows.at[dsl], sem))
                copies.append(pltpu.async_copy(rel.at[ridx.at[isl]], rrows.at[dsl], sem))
                copies.append(pltpu.async_copy(ent2.at[tpi.at[isl]], trows.at[dsl], sem))
            for cp in copies:
                cp.wait()

            def pbody(g, carry):
                sl = pl.ds(hbase + g * LANES, LANES)
                rows16 = g * LANES + lax.iota(jnp.int32, LANES)
                for idxref, rowsref in ((hidx, hrows), (tidx, trows)):
                    e = idxref[sl]
                    m = e >= NMAIN
                    n = plsc.all_reduce_population_count(m)

                    @pl.when(jnp.max(n) > 0)
                    def _patch():
                        ce = (e - NMAIN) & (TAILN - 1)
                        off = jnp.where(e >= HALF_OFF, EMBED, 0)
                        for jj in range(EMBED):
                            cj = jnp.full((LANES,), jj, jnp.int32)
                            cv = plsc.load_gather(cbuf, [ce, cj])
                            plsc.store_scatter(rowsref, [rows16, off + cj], cv, mask=m)

                return carry

            lax.fori_loop(0, ngroups, pbody, 0)

            def gbody(g, carry):
                rows16 = g * LANES + lax.iota(jnp.int32, LANES)
                sl = pl.ds(hbase + g * LANES, LANES)
                hoff = jnp.where(hidx[sl] >= HALF_OFF, EMBED, 0)
                toff = jnp.where(tidx[sl] >= HALF_OFF, EMBED, 0)

                def jbody(j, acc):
                    for u in range(4):
                        col = jnp.full((LANES,), j * 4 + u, jnp.int32)
                        hv = plsc.load_gather(hrows, [rows16, hoff + col])
                        rv = plsc.load_gather(rrows, [rows16, col])
                        tv = plsc.load_gather(trows, [rows16, toff + col])
                        d = (hv + rv) - tv
                        acc = acc + d * d
                    return acc

                acc = lax.fori_loop(
                    0, EMBED // 4, jbody, jnp.zeros((LANES,), jnp.float32)
                )
                outv[pl.ds(hbase + g * LANES, LANES)] = _sqrt16(acc)
                return carry

            lax.fori_loop(0, ngroups, gbody, 0)

        pltpu.sync_copy(outv, out.at[pl.ds(base, bpt)])

    return trans_e


def kernel(entity_embeddings, relation_embeddings, head, relation, tail):
    info = plsc.get_sparse_core_info()
    nc, ns = info.num_cores, info.num_subcores
    batch = head.shape[0]
    bpt = batch // (nc * ns)
    entT = entity_embeddings.T
    eye = jnp.eye(EMBED, dtype=jnp.float32)
    ent2 = _tpose(nc, ns)(entT, entT, eye)
    corner = entity_embeddings[NMAIN:, :]
    out = _build(nc, ns, bpt)(
        ent2, relation_embeddings, corner, head, relation, tail
    )
    return out


# speedup vs baseline: 6.3526x; 1.2298x over previous
import functools

import jax
import jax.numpy as jnp
from jax import lax
from jax.experimental import pallas as pl
from jax.experimental.pallas import tpu as pltpu
from jax.experimental.pallas import tpu_sc as plsc

EMBED = 64
LANES = 16
NENT = 1000000
TAILN = 64
NMAIN = NENT - TAILN
HALF_OFF = NMAIN // 2
TBLK = 8064
NBLK = HALF_OFF // TBLK
HALF = 256


def _tpose_kernel(a_ref, b_ref, eye_ref, o_ref):
    dn = (((0,), (0,)), ((), ()))
    at = lax.dot_general(a_ref[...], eye_ref[...], dn,
                         preferred_element_type=jnp.float32)
    bt = lax.dot_general(b_ref[...], eye_ref[...], dn,
                         preferred_element_type=jnp.float32)
    o_ref[...] = jnp.concatenate([at, bt], axis=1)


@functools.lru_cache(maxsize=None)
def _tpose(nc, ns):
    del nc, ns
    return pl.pallas_call(
        _tpose_kernel,
        grid=(NBLK,),
        in_specs=[
            pl.BlockSpec((EMBED, TBLK), lambda j: (0, j)),
            pl.BlockSpec((EMBED, TBLK), lambda j: (0, j + NBLK)),
            pl.BlockSpec((EMBED, EMBED), lambda j: (0, 0)),
        ],
        out_specs=pl.BlockSpec((TBLK, 128), lambda j: (j, 0)),
        out_shape=jax.ShapeDtypeStruct((HALF_OFF, 128), jnp.float32),
    )


def _sqrt16(a):
    i = lax.bitcast_convert_type(a, jnp.int32)
    y = lax.bitcast_convert_type(jnp.int32(0x5F3759DF) - (i >> 1), jnp.float32)
    for _ in range(3):
        y = y * (1.5 - (0.5 * a * y) * y)
    return a * y


@functools.lru_cache(maxsize=None)
def _build(nc, ns, bpt):
    ngroups = HALF // LANES
    mesh = plsc.VectorSubcoreMesh(core_axis_name="c", subcore_axis_name="s")

    @functools.partial(
        pl.kernel,
        out_type=jax.ShapeDtypeStruct((nc * ns * bpt,), jnp.float32),
        mesh=mesh,
        compiler_params=pltpu.CompilerParams(
            needs_layout_passes=False, use_tc_tiling_on_sc=False
        ),
        scratch_types=[
            pltpu.VMEM((bpt,), jnp.int32),
            pltpu.VMEM((bpt,), jnp.int32),
            pltpu.VMEM((bpt,), jnp.int32),
            pltpu.VMEM((HALF, 128), jnp.float32),
            pltpu.VMEM((HALF, EMBED), jnp.float32),
            pltpu.VMEM((HALF, 128), jnp.float32),
            pltpu.VMEM((TAILN, EMBED), jnp.float32),
            pltpu.VMEM((bpt,), jnp.float32),
            pltpu.VMEM((bpt,), jnp.int32),
            pltpu.VMEM((bpt,), jnp.int32),
            pltpu.SemaphoreType.DMA,
        ],
    )
    def trans_e(ent2, rel, corner, headi, reli, taili, out,
                hidx, ridx, tidx, hrows, rrows, trows, cbuf, outv, hpi, tpi, sem):
        c = lax.axis_index("c")
        s = lax.axis_index("s")
        wid = s * nc + c
        base = wid * bpt

        pltpu.sync_copy(headi.at[pl.ds(base, bpt)], hidx)
        pltpu.sync_copy(reli.at[pl.ds(base, bpt)], ridx)
        pltpu.sync_copy(taili.at[pl.ds(base, bpt)], tidx)
        pltpu.sync_copy(corner, cbuf)

        def ibody(g, carry):
            sl = pl.ds(g * LANES, LANES)
            for src, dst in ((hidx, hpi), (tidx, tpi)):
                e = src[sl]
                pi = jnp.where(e >= HALF_OFF, e - HALF_OFF, e)
                dst[sl] = jnp.where(e >= NMAIN, 0, pi)
            return carry

        lax.fori_loop(0, bpt // LANES, ibody, 0)

        for half in range(bpt // HALF):
            hbase = half * HALF
            copies = []
            for j in range(HALF // 128):
                isl = pl.ds(hbase + j * 128, 128)
                dsl = pl.ds(j * 128, 128)
                copies.append(pltpu.async_copy(ent2.at[hpi.at[isl]], hrows.at[dsl], sem))
                copies.append(pltpu.async_copy(rel.at[ridx.at[isl]], rrows.at[dsl], sem))
                copies.append(pltpu.async_copy(ent2.at[tpi.at[isl]], trows.at[dsl], sem))
            for cp in copies:
                cp.wait()

            def pbody(g, carry):
                sl = pl.ds(hbase + g * LANES, LANES)
                rows16 = g * LANES + lax.iota(jnp.int32, LANES)
                for idxref, rowsref in ((hidx, hrows), (tidx, trows)):
                    e = idxref[sl]
                    m = e >= NMAIN
                    n = plsc.all_reduce_population_count(m)

                    @pl.when(jnp.max(n) > 0)
                    def _patch():
                        ce = (e - NMAIN) & (TAILN - 1)
                        off = jnp.where(e >= HALF_OFF, EMBED, 0)
                        for jj in range(EMBED):
                            cj = jnp.full((LANES,), jj, jnp.int32)
                            cv = plsc.load_gather(cbuf, [ce, cj])
                            plsc.store_scatter(rowsref, [rows16, off + cj], cv, mask=m)

                return carry

            lax.fori_loop(0, ngroups, pbody, 0)

            def gbody(g, carry):
                rows16 = g * LANES + lax.iota(jnp.int32, LANES)
                sl = pl.ds(hbase + g * LANES, LANES)
                hoff = jnp.where(hidx[sl] >= HALF_OFF, EMBED, 0)
                toff = jnp.where(tidx[sl] >= HALF_OFF, EMBED, 0)

                def jbody(j, acc):
                    for u in range(4):
                        col = jnp.full((LANES,), j * 4 + u, jnp.int32)
                        hv = plsc.load_gather(hrows, [rows16, hoff + col])
                        rv = plsc.load_gather(rrows, [rows16, col])
                        tv = plsc.load_gather(trows, [rows16, toff + col])
                        d = (hv + rv) - tv
                        acc = acc + d * d
                    return acc

                acc = lax.fori_loop(
                    0, EMBED // 4, jbody, jnp.zeros((LANES,), jnp.float32)
                )
                outv[pl.ds(hbase + g * LANES, LANES)] = _sqrt16(acc)
                return carry

            lax.fori_loop(0, ngroups, gbody, 0)

        pltpu.sync_copy(outv, out.at[pl.ds(base, bpt)])

    return trans_e


def kernel(entity_embeddings, relation_embeddings, head, relation, tail):
    info = plsc.get_sparse_core_info()
    nc, ns = info.num_cores, info.num_subcores
    batch = head.shape[0]
    bpt = batch // (nc * ns)
    entT = entity_embeddings.T
    eye = jnp.eye(EMBED, dtype=jnp.float32)
    ent2 = _tpose(nc, ns)(entT, entT, eye)
    corner = entity_embeddings[NMAIN:, :]
    out = _build(nc, ns, bpt)(
        ent2, relation_embeddings, corner, head, relation, tail
    )
    return out


# speedup vs baseline: 6.6738x; 1.0506x over previous
import functools

import jax
import jax.numpy as jnp
from jax import lax
from jax.experimental import pallas as pl
from jax.experimental.pallas import tpu as pltpu
from jax.experimental.pallas import tpu_sc as plsc

EMBED = 64
LANES = 16
NENT = 1000000
TAILN = 64
NMAIN = NENT - TAILN
HALF_OFF = NMAIN // 2
TBLK = 16128
NBLK = HALF_OFF // TBLK
HALF = 256


def _tpose_kernel(a_ref, b_ref, eye_ref, o_ref):
    dn = (((0,), (0,)), ((), ()))
    at = lax.dot_general(a_ref[...], eye_ref[...], dn,
                         preferred_element_type=jnp.float32)
    bt = lax.dot_general(b_ref[...], eye_ref[...], dn,
                         preferred_element_type=jnp.float32)
    o_ref[...] = jnp.concatenate([at, bt], axis=1)


@functools.lru_cache(maxsize=None)
def _tpose(nc, ns):
    del nc, ns
    return pl.pallas_call(
        _tpose_kernel,
        grid=(NBLK,),
        in_specs=[
            pl.BlockSpec((EMBED, TBLK), lambda j: (0, j)),
            pl.BlockSpec((EMBED, TBLK), lambda j: (0, j + NBLK)),
            pl.BlockSpec((EMBED, EMBED), lambda j: (0, 0)),
        ],
        out_specs=pl.BlockSpec((TBLK, 128), lambda j: (j, 0)),
        out_shape=jax.ShapeDtypeStruct((HALF_OFF, 128), jnp.float32),
    )


def _sqrt16(a):
    i = lax.bitcast_convert_type(a, jnp.int32)
    y = lax.bitcast_convert_type(jnp.int32(0x5F3759DF) - (i >> 1), jnp.float32)
    for _ in range(3):
        y = y * (1.5 - (0.5 * a * y) * y)
    return a * y


@functools.lru_cache(maxsize=None)
def _build(nc, ns, bpt):
    ngroups = HALF // LANES
    mesh = plsc.VectorSubcoreMesh(core_axis_name="c", subcore_axis_name="s")

    @functools.partial(
        pl.kernel,
        out_type=jax.ShapeDtypeStruct((nc * ns * bpt,), jnp.float32),
        mesh=mesh,
        compiler_params=pltpu.CompilerParams(
            needs_layout_passes=False, use_tc_tiling_on_sc=False
        ),
        scratch_types=[
            pltpu.VMEM((bpt,), jnp.int32),
            pltpu.VMEM((bpt,), jnp.int32),
            pltpu.VMEM((bpt,), jnp.int32),
            pltpu.VMEM((HALF, 128), jnp.float32),
            pltpu.VMEM((HALF, EMBED), jnp.float32),
            pltpu.VMEM((HALF, 128), jnp.float32),
            pltpu.VMEM((TAILN, EMBED), jnp.float32),
            pltpu.VMEM((bpt,), jnp.float32),
            pltpu.VMEM((bpt,), jnp.int32),
            pltpu.VMEM((bpt,), jnp.int32),
            pltpu.SemaphoreType.DMA,
        ],
    )
    def trans_e(ent2, rel, corner, headi, reli, taili, out,
                hidx, ridx, tidx, hrows, rrows, trows, cbuf, outv, hpi, tpi, sem):
        c = lax.axis_index("c")
        s = lax.axis_index("s")
        wid = s * nc + c
        base = wid * bpt

        pltpu.sync_copy(headi.at[pl.ds(base, bpt)], hidx)
        pltpu.sync_copy(reli.at[pl.ds(base, bpt)], ridx)
        pltpu.sync_copy(taili.at[pl.ds(base, bpt)], tidx)
        pltpu.sync_copy(corner, cbuf)

        def ibody(g, carry):
            sl = pl.ds(g * LANES, LANES)
            for src, dst in ((hidx, hpi), (tidx, tpi)):
                e = src[sl]
                pi = jnp.where(e >= HALF_OFF, e - HALF_OFF, e)
                dst[sl] = jnp.where(e >= NMAIN, 0, pi)
            return carry

        lax.fori_loop(0, bpt // LANES, ibody, 0)

        for half in range(bpt // HALF):
            hbase = half * HALF
            copies = []
            for j in range(HALF // 128):
                isl = pl.ds(hbase + j * 128, 128)
                dsl = pl.ds(j * 128, 128)
                copies.append(pltpu.async_copy(ent2.at[hpi.at[isl]], hrows.at[dsl], sem))
                copies.append(pltpu.async_copy(rel.at[ridx.at[isl]], rrows.at[dsl], sem))
                copies.append(pltpu.async_copy(ent2.at[tpi.at[isl]], trows.at[dsl], sem))
            for cp in copies:
                cp.wait()

            def pbody(g, carry):
                sl = pl.ds(hbase + g * LANES, LANES)
                rows16 = g * LANES + lax.iota(jnp.int32, LANES)
                for idxref, rowsref in ((hidx, hrows), (tidx, trows)):
                    e = idxref[sl]
                    m = e >= NMAIN
                    n = plsc.all_reduce_population_count(m)

                    @pl.when(jnp.max(n) > 0)
                    def _patch():
                        ce = (e - NMAIN) & (TAILN - 1)
                        off = jnp.where(e >= HALF_OFF, EMBED, 0)
                        for jj in range(EMBED):
                            cj = jnp.full((LANES,), jj, jnp.int32)
                            cv = plsc.load_gather(cbuf, [ce, cj])
                            plsc.store_scatter(rowsref, [rows16, off + cj], cv, mask=m)

                return carry

            lax.fori_loop(0, ngroups, pbody, 0)

            def gbody(g, carry):
                rows16 = g * LANES + lax.iota(jnp.int32, LANES)
                sl = pl.ds(hbase + g * LANES, LANES)
                hoff = jnp.where(hidx[sl] >= HALF_OFF, EMBED, 0)
                toff = jnp.where(tidx[sl] >= HALF_OFF, EMBED, 0)

                def jbody(j, acc):
                    for u in range(4):
                        col = jnp.full((LANES,), j * 4 + u, jnp.int32)
                        hv = plsc.load_gather(hrows, [rows16, hoff + col])
                        rv = plsc.load_gather(rrows, [rows16, col])
                        tv = plsc.load_gather(trows, [rows16, toff + col])
                        d = (hv + rv) - tv
                        acc = acc + d * d
                    return acc

                acc = lax.fori_loop(
                    0, EMBED // 4, jbody, jnp.zeros((LANES,), jnp.float32)
                )
                outv[pl.ds(hbase + g * LANES, LANES)] = _sqrt16(acc)
                return carry

            lax.fori_loop(0, ngroups, gbody, 0)

        pltpu.sync_copy(outv, out.at[pl.ds(base, bpt)])

    return trans_e


def kernel(entity_embeddings, relation_embeddings, head, relation, tail):
    info = plsc.get_sparse_core_info()
    nc, ns = info.num_cores, info.num_subcores
    batch = head.shape[0]
    bpt = batch // (nc * ns)
    entT = entity_embeddings.T
    eye = jnp.eye(EMBED, dtype=jnp.float32)
    ent2 = _tpose(nc, ns)(entT, entT, eye)
    corner = entity_embeddings[NMAIN:, :]
    out = _build(nc, ns, bpt)(
        ent2, relation_embeddings, corner, head, relation, tail
    )
    return out
